# baseline (speedup 1.0000x reference)
import functools

import jax
import jax.numpy as jnp
from jax import lax
from jax.experimental import pallas as pl
from jax.experimental.pallas import tpu as pltpu
from jax.experimental.pallas import tpu_sc as plsc

_VOXEL_X = 0.005
_VOXEL_Y = 0.005
_PC_X = 0.0
_PC_Y = 0.0

_B = 4
_NK = 4096
_C = 256
_H = 200
_W = 200
_HW = _H * _W
_L = 16
_NWORK = 32
_NPPW = (_B * _NK) // _NWORK
_WPB = _NK // _NPPW
_PCH = 32
_NCHUNK = _NPPW // _PCH
_RING = 2
_CCH = _C // _L


def _sc_body(tab_hbm, kpx_hbm, kpy_hbm, stride_hbm, out_hbm,
             kx_v, ky_v, sv_v,
             ia_v, ib_v,
             wa_v, wb_v, wc_v, wd_v,
             rows_v, ob_v, sem, osem):
    wid = lax.axis_index("s") * 2 + lax.axis_index("c")
    b = wid // _WPB
    q0 = (wid % _WPB) * _NPPW

    pltpu.sync_copy(kpx_hbm.at[b, pl.ds(q0, _NPPW)], kx_v)
    pltpu.sync_copy(kpy_hbm.at[b, pl.ds(q0, _NPPW)], ky_v)
    pltpu.sync_copy(stride_hbm, sv_v)
    stride = sv_v[...]
    rbase = b * _HW

    def prep(i):
        sl = pl.ds(i * _L, _L)
        x = ((kx_v[sl] - _PC_X) / _VOXEL_X) / stride
        y = ((ky_v[sl] - _PC_Y) / _VOXEL_Y) / stride
        xt = x.astype(jnp.int32)
        x0 = jnp.where(x < xt.astype(jnp.float32), xt - 1, xt)
        yt = y.astype(jnp.int32)
        y0 = jnp.where(y < yt.astype(jnp.float32), yt - 1, yt)
        x0c = jnp.clip(x0, 0, _W - 1)
        x1c = jnp.clip(x0 + 1, 0, _W - 1)
        y0c = jnp.clip(y0, 0, _H - 1)
        y1c = jnp.clip(y0 + 1, 0, _H - 1)
        x0f = x0c.astype(jnp.float32)
        x1f = x1c.astype(jnp.float32)
        y0f = y0c.astype(jnp.float32)
        y1f = y1c.astype(jnp.float32)
        cb = (i // 2) * (2 * _PCH) + (i % 2) * _L
        ia_v[pl.ds(cb, _L)] = y0c * _W + x0c + rbase
        ia_v[pl.ds(cb + _PCH, _L)] = y0c * _W + x1c + rbase
        ib_v[pl.ds(cb, _L)] = y1c * _W + x0c + rbase
        ib_v[pl.ds(cb + _PCH, _L)] = y1c * _W + x1c + rbase
        wa_v[sl] = (x1f - x) * (y1f - y)
        wb_v[sl] = (x1f - x) * (y - y0f)
        wc_v[sl] = (x - x0f) * (y1f - y)
        wd_v[sl] = (x - x0f) * (y - y0f)

    def start_gathers(c, k):
        sl = pl.ds(c * 2 * _PCH, 2 * _PCH)
        pltpu.async_copy(tab_hbm.at[ia_v.at[sl]], rows_v.at[k, pl.ds(0, 2 * _PCH)], sem)
        pltpu.async_copy(tab_hbm.at[ib_v.at[sl]], rows_v.at[k, pl.ds(2 * _PCH, 2 * _PCH)], sem)

    def wait_gathers(c, k):
        del c
        pltpu.make_async_copy(tab_hbm.at[pl.ds(0, 4 * _PCH)], rows_v.at[k], sem).wait()

    def out_slice(c):
        return out_hbm.at[b, pl.ds(q0 + c * _PCH, _PCH)]

    def prep_loop(i, carry):
        prep(i)
        return carry

    lax.fori_loop(0, _NPPW // _L, prep_loop, 0)
    start_gathers(0, 0)

    def chunk(g, carry):
        for k in (0, 1):
            c = g * 2 + k
            cn = jnp.minimum(c + 1, _NCHUNK - 1)
            start_gathers(cn, (k + 1) % 2)
            wait_gathers(c, k)

            @pl.when(g >= 1)
            def _():
                pltpu.make_async_copy(ob_v.at[k], out_slice(c - 2), osem).wait()

            def point(p, inner):
                q = c * _PCH + p
                wa = wa_v[pl.ds(q, _L)][0]
                wb = wb_v[pl.ds(q, _L)][0]
                wc = wc_v[pl.ds(q, _L)][0]
                wd = wd_v[pl.ds(q, _L)][0]
                for j in range(_CCH):
                    sl = pl.ds(j * _L, _L)
                    ob_v[k, p, sl] = (rows_v[k, p, sl] * wa
                                      + rows_v[k, _PCH + p, sl] * wc
                                      + rows_v[k, 2 * _PCH + p, sl] * wb
                                      + rows_v[k, 3 * _PCH + p, sl] * wd)
                return inner

            lax.fori_loop(0, _PCH, point, 0)
            pltpu.async_copy(ob_v.at[k], out_slice(c), osem)
        return carry

    lax.fori_loop(0, _NCHUNK // 2, chunk, 0)
    wait_gathers(_NCHUNK - 1, 0)
    pltpu.make_async_copy(ob_v.at[0], out_slice(_NCHUNK - 2), osem).wait()
    pltpu.make_async_copy(ob_v.at[1], out_slice(_NCHUNK - 1), osem).wait()


_sc_interp = functools.partial(
    pl.kernel,
    mesh=plsc.VectorSubcoreMesh(core_axis_name="c", subcore_axis_name="s"),
    compiler_params=pltpu.CompilerParams(needs_layout_passes=False),
    out_type=jax.ShapeDtypeStruct((_B, _NK, _C), jnp.float32),
    scratch_types=[
        pltpu.VMEM((_NPPW,), jnp.float32),
        pltpu.VMEM((_NPPW,), jnp.float32),
        pltpu.VMEM((_L,), jnp.float32),
        pltpu.VMEM((2 * _NPPW,), jnp.int32),
        pltpu.VMEM((2 * _NPPW,), jnp.int32),
        pltpu.VMEM((_NPPW + _L,), jnp.float32),
        pltpu.VMEM((_NPPW + _L,), jnp.float32),
        pltpu.VMEM((_NPPW + _L,), jnp.float32),
        pltpu.VMEM((_NPPW + _L,), jnp.float32),
        pltpu.VMEM((2, 4 * _PCH, _C), jnp.float32),
        pltpu.VMEM((2, _PCH, _C), jnp.float32),
        pltpu.SemaphoreType.DMA,
        pltpu.SemaphoreType.DMA,
    ],
)(_sc_body)


def kernel(keypoints, bev_features, bev_stride):
    kpx = keypoints[:, :, 0]
    kpy = keypoints[:, :, 1]
    tab = jnp.transpose(bev_features, (0, 2, 3, 1)).reshape(_B * _HW, _C)
    stride_vec = jnp.full((_L,), bev_stride, jnp.float32)
    return _sc_interp(tab, kpx, kpy, stride_vec)


# speedup vs baseline: 1.0087x; 1.0087x over previous
import functools

import jax
import jax.numpy as jnp
from jax import lax
from jax.experimental import pallas as pl
from jax.experimental.pallas import tpu as pltpu
from jax.experimental.pallas import tpu_sc as plsc

_VOXEL_X = 0.005
_VOXEL_Y = 0.005
_PC_X = 0.0
_PC_Y = 0.0

_B = 4
_NK = 4096
_C = 256
_H = 200
_W = 200
_HW = _H * _W
_L = 16
_NWORK = 32
_NPPW = (_B * _NK) // _NWORK
_WPB = _NK // _NPPW
_PCH = 32
_NCHUNK = _NPPW // _PCH
_RING = 2
_CCH = _C // _L


def _sc_body(tab_hbm, kpx_hbm, kpy_hbm, stride_hbm, out_hbm,
             kx_v, ky_v, sv_v,
             ia_v, ib_v, ic_v, id_v,
             wa_v, wb_v, wc_v, wd_v,
             rows_v, ob_v, sem, osem):
    wid = lax.axis_index("s") * 2 + lax.axis_index("c")
    b = wid // _WPB
    q0 = (wid % _WPB) * _NPPW

    pltpu.sync_copy(kpx_hbm.at[b, pl.ds(q0, _NPPW)], kx_v)
    pltpu.sync_copy(kpy_hbm.at[b, pl.ds(q0, _NPPW)], ky_v)
    pltpu.sync_copy(stride_hbm, sv_v)
    stride = sv_v[...]
    rbase = b * _HW

    def prep(i):
        sl = pl.ds(i * _L, _L)
        x = ((kx_v[sl] - _PC_X) / _VOXEL_X) / stride
        y = ((ky_v[sl] - _PC_Y) / _VOXEL_Y) / stride
        xt = x.astype(jnp.int32)
        x0 = jnp.where(x < xt.astype(jnp.float32), xt - 1, xt)
        yt = y.astype(jnp.int32)
        y0 = jnp.where(y < yt.astype(jnp.float32), yt - 1, yt)
        x0c = jnp.clip(x0, 0, _W - 1)
        x1c = jnp.clip(x0 + 1, 0, _W - 1)
        y0c = jnp.clip(y0, 0, _H - 1)
        y1c = jnp.clip(y0 + 1, 0, _H - 1)
        x0f = x0c.astype(jnp.float32)
        x1f = x1c.astype(jnp.float32)
        y0f = y0c.astype(jnp.float32)
        y1f = y1c.astype(jnp.float32)
        ia_v[sl] = y0c * _W + x0c + rbase
        ib_v[sl] = y1c * _W + x0c + rbase
        ic_v[sl] = y0c * _W + x1c + rbase
        id_v[sl] = y1c * _W + x1c + rbase
        wa_v[sl] = (x1f - x) * (y1f - y)
        wb_v[sl] = (x1f - x) * (y - y0f)
        wc_v[sl] = (x - x0f) * (y1f - y)
        wd_v[sl] = (x - x0f) * (y - y0f)

    def start_gathers(c, k):
        sl = pl.ds(c * _PCH, _PCH)
        pltpu.async_copy(tab_hbm.at[ia_v.at[sl]], rows_v.at[k, pl.ds(0, _PCH)], sem)
        pltpu.async_copy(tab_hbm.at[ic_v.at[sl]], rows_v.at[k, pl.ds(_PCH, _PCH)], sem)
        pltpu.async_copy(tab_hbm.at[ib_v.at[sl]], rows_v.at[k, pl.ds(2 * _PCH, _PCH)], sem)
        pltpu.async_copy(tab_hbm.at[id_v.at[sl]], rows_v.at[k, pl.ds(3 * _PCH, _PCH)], sem)

    def wait_gathers(c, k):
        del c
        pltpu.make_async_copy(tab_hbm.at[pl.ds(0, 4 * _PCH)], rows_v.at[k], sem).wait()

    def out_slice(c):
        return out_hbm.at[b, pl.ds(q0 + c * _PCH, _PCH)]

    def prep_loop(i, carry):
        prep(i)
        return carry

    lax.fori_loop(0, _NPPW // _L, prep_loop, 0)
    start_gathers(0, 0)

    def chunk(g, carry):
        for k in (0, 1):
            c = g * 2 + k
            cn = jnp.minimum(c + 1, _NCHUNK - 1)
            start_gathers(cn, (k + 1) % 2)
            wait_gathers(c, k)

            @pl.when(g >= 1)
            def _():
                pltpu.make_async_copy(ob_v.at[k], out_slice(c - 2), osem).wait()

            def point(p, inner):
                q = c * _PCH + p
                wa = wa_v[pl.ds(q, _L)][0]
                wb = wb_v[pl.ds(q, _L)][0]
                wc = wc_v[pl.ds(q, _L)][0]
                wd = wd_v[pl.ds(q, _L)][0]
                for j in range(_CCH):
                    sl = pl.ds(j * _L, _L)
                    ob_v[k, p, sl] = (rows_v[k, p, sl] * wa
                                      + rows_v[k, _PCH + p, sl] * wc
                                      + rows_v[k, 2 * _PCH + p, sl] * wb
                                      + rows_v[k, 3 * _PCH + p, sl] * wd)
                return inner

            lax.fori_loop(0, _PCH, point, 0)
            pltpu.async_copy(ob_v.at[k], out_slice(c), osem)
        return carry

    lax.fori_loop(0, _NCHUNK // 2, chunk, 0)
    wait_gathers(_NCHUNK - 1, 0)
    pltpu.make_async_copy(ob_v.at[0], out_slice(_NCHUNK - 2), osem).wait()
    pltpu.make_async_copy(ob_v.at[1], out_slice(_NCHUNK - 1), osem).wait()


_sc_interp = functools.partial(
    pl.kernel,
    mesh=plsc.VectorSubcoreMesh(core_axis_name="c", subcore_axis_name="s"),
    compiler_params=pltpu.CompilerParams(needs_layout_passes=False),
    out_type=jax.ShapeDtypeStruct((_B, _NK, _C), jnp.float32),
    scratch_types=[
        pltpu.VMEM((_NPPW,), jnp.float32),
        pltpu.VMEM((_NPPW,), jnp.float32),
        pltpu.VMEM((_L,), jnp.float32),
        pltpu.VMEM((_NPPW,), jnp.int32),
        pltpu.VMEM((_NPPW,), jnp.int32),
        pltpu.VMEM((_NPPW,), jnp.int32),
        pltpu.VMEM((_NPPW,), jnp.int32),
        pltpu.VMEM((_NPPW + _L,), jnp.float32),
        pltpu.VMEM((_NPPW + _L,), jnp.float32),
        pltpu.VMEM((_NPPW + _L,), jnp.float32),
        pltpu.VMEM((_NPPW + _L,), jnp.float32),
        pltpu.VMEM((2, 4 * _PCH, _C), jnp.float32),
        pltpu.VMEM((2, _PCH, _C), jnp.float32),
        pltpu.SemaphoreType.DMA,
        pltpu.SemaphoreType.DMA,
    ],
)(_sc_body)


def kernel(keypoints, bev_features, bev_stride):
    kpx = keypoints[:, :, 0]
    kpy = keypoints[:, :, 1]
    tab = jnp.transpose(bev_features, (0, 2, 3, 1)).reshape(_B * _HW, _C)
    stride_vec = jnp.full((_L,), bev_stride, jnp.float32)
    return _sc_interp(tab, kpx, kpy, stride_vec)
